# baseline (speedup 1.0000x reference)
import functools

import jax
import jax.numpy as jnp
from jax.experimental import pallas as pl
from jax.experimental.pallas import tpu as pltpu

N_BLK = 1024


def _mlp_body(eps_ref, h_ref, agg_ref, w1_ref, b1_ref, w2_ref, b2_ref, out_ref):
    h = h_ref[...]
    agg = agg_ref[...]
    z = (1.0 + eps_ref[0]) * h + agg
    z = jnp.maximum(jnp.dot(z, w1_ref[...], preferred_element_type=jnp.float32)
                    + b1_ref[...], 0.0)
    z = jnp.dot(z, w2_ref[...], preferred_element_type=jnp.float32) + b2_ref[...]
    out_ref[...] = jnp.maximum(z, 0.0)


def _mlp_layer(eps_l, h, agg, w1, b1, w2, b2):
    n = h.shape[0]
    grid = (n // N_BLK,)
    return pl.pallas_call(
        _mlp_body,
        grid=grid,
        in_specs=[
            pl.BlockSpec(memory_space=pltpu.SMEM),
            pl.BlockSpec((N_BLK, 64), lambda i: (i, 0)),
            pl.BlockSpec((N_BLK, 64), lambda i: (i, 0)),
            pl.BlockSpec((64, 64), lambda i: (0, 0)),
            pl.BlockSpec((64,), lambda i: (0,)),
            pl.BlockSpec((64, 64), lambda i: (0, 0)),
            pl.BlockSpec((64,), lambda i: (0,)),
        ],
        out_specs=pl.BlockSpec((N_BLK, 64), lambda i: (i, 0)),
        out_shape=jax.ShapeDtypeStruct((n, 64), jnp.float32),
    )(eps_l.reshape(1), h, agg, w1, b1, w2, b2)


def _head_body(pool_ref, cnt_ref, w1_ref, b1_ref, w2f_ref, sel_ref, b2_ref, out_ref):
    g = pool_ref[...] / jnp.maximum(cnt_ref[...], 1.0)
    t = jnp.maximum(jnp.dot(g, w1_ref[...], preferred_element_type=jnp.float32)
                    + b1_ref[...], 0.0)
    u = t * w2f_ref[...]
    out_ref[...] = jnp.dot(u, sel_ref[...], preferred_element_type=jnp.float32) + b2_ref[...]


def _head(pool, cnt, w1cat, b1cat, w2flat, sel, b2row):
    B = pool.shape[0]
    P = sel.shape[1]
    return pl.pallas_call(
        _head_body,
        out_shape=jax.ShapeDtypeStruct((B, P), jnp.float32),
    )(pool, cnt, w1cat, b1cat, w2flat, sel, b2row)


def kernel(x, edge_index, batch, embed, CW1, Cb1, CW2, Cb2, eps, HW1, Hb1, HW2, Hb2):
    src = edge_index[0]
    dst = edge_index[1]
    n = x.shape[0]
    n_pad = ((n + N_BLK - 1) // N_BLK) * N_BLK
    h = jnp.take(embed, x, axis=0)
    h = jnp.pad(h, ((0, n_pad - n), (0, 0)))
    for l in range(3):
        agg = jnp.zeros_like(h).at[dst].add(jnp.take(h, src, axis=0))
        h = _mlp_layer(eps[l], h, agg, CW1[l], Cb1[l], CW2[l], Cb2[l])
    B = 256
    sums = jax.ops.segment_sum(h[:n], batch, num_segments=B)
    cnt = jax.ops.segment_sum(jnp.ones((n, 1), dtype=h.dtype), batch,
                              num_segments=B)
    P, H = HW1.shape[0], HW1.shape[1]
    w1cat = HW1.transpose(1, 0, 2).reshape(H, P * H)
    b1cat = Hb1.reshape(P * H)
    w2flat = HW2[:, :, 0].reshape(P * H)
    sel = jnp.repeat(jnp.eye(P, dtype=jnp.float32), H, axis=0)
    b2row = Hb2[:, 0]
    return _head(sums, cnt, w1cat, b1cat, w2flat, sel, b2row)


# speedup vs baseline: 3.8862x; 3.8862x over previous
import functools

import jax
import jax.numpy as jnp
from jax import lax
from jax.experimental import pallas as pl
from jax.experimental.pallas import tpu as pltpu
from jax.experimental.pallas import tpu_sc as plsc

N = 50000
E = 800000
B = 256
H = 64
V = 51

NC = 2
NS = 16
CE = 128

N_BLK = 1024
N_PAD = 51200
NODE_T = N_PAD // NS
EDGE_C = -(-E // (NS * CE))
E_T = EDGE_C * CE
E_PAD = NS * E_T
AGG_ROWS = N_PAD + CE
ZERO_T = AGG_ROWS // NS
PB = B + 8

_MESH = plsc.VectorSubcoreMesh(core_axis_name="c", subcore_axis_name="s")


def _embed_body(x2, table, out, idx_v, rows_v, sem):
    c = lax.axis_index("c")
    s = lax.axis_index("s")

    def step(t, _):
        base = s * NODE_T + t * CE
        pltpu.sync_copy(x2.at[c, pl.ds(base, CE)], idx_v)
        pltpu.async_copy(table.at[idx_v], rows_v, sem).wait()
        pltpu.sync_copy(rows_v, out.at[c, pl.ds(base, CE)])
        return _

    lax.fori_loop(0, NODE_T // CE, step, 0)


def _embed(x2, table):
    return pl.kernel(
        _embed_body,
        out_type=jax.ShapeDtypeStruct((NC, N_PAD, 32), jnp.float32),
        mesh=_MESH,
        compiler_params=pltpu.CompilerParams(use_tc_tiling_on_sc=False),
        scratch_types=[
            pltpu.VMEM((CE,), jnp.int32),
            pltpu.VMEM((CE, 32), jnp.float32),
            pltpu.SemaphoreType.DMA,
        ],
    )(x2, table)


def _agg_body(h2d, src2, dstp, zrows, out, agg_s, sidx, didx, rows_v, sem):
    c = lax.axis_index("c")
    s = lax.axis_index("s")
    pltpu.sync_copy(zrows.at[pl.ds(s * ZERO_T, ZERO_T)],
                    agg_s.at[pl.ds(s * ZERO_T, ZERO_T)])
    plsc.subcore_barrier()

    def step(t, _):
        e0 = s * E_T + t * CE
        pltpu.sync_copy(src2.at[c, pl.ds(e0, CE)], sidx)
        pltpu.sync_copy(dstp.at[pl.ds(e0, CE)], didx)
        pltpu.async_copy(h2d.at[sidx], rows_v, sem).wait()
        pltpu.sync_copy(rows_v, agg_s.at[didx], add=True)
        return _

    lax.fori_loop(0, EDGE_C, step, 0)
    plsc.subcore_barrier()
    pltpu.sync_copy(agg_s.at[pl.ds(s * NODE_T, NODE_T)],
                    out.at[c, pl.ds(s * NODE_T, NODE_T)])


def _agg(h2d, src2, dstp, zrows):
    return pl.kernel(
        _agg_body,
        out_type=jax.ShapeDtypeStruct((NC, N_PAD, 32), jnp.float32),
        mesh=_MESH,
        compiler_params=pltpu.CompilerParams(use_tc_tiling_on_sc=False),
        scratch_types=[
            pltpu.VMEM_SHARED((AGG_ROWS, 32), jnp.float32),
            pltpu.VMEM((CE,), jnp.int32),
            pltpu.VMEM((CE,), jnp.int32),
            pltpu.VMEM((CE, 32), jnp.float32),
            pltpu.SemaphoreType.DMA,
        ],
    )(h2d, src2, dstp, zrows)


def _pool_body(h, batchp, ones_h, zpool, zcnt, outp, outc,
               pool_s, cnt_s, bidx, rows_v, ones_v, sem):
    c = lax.axis_index("c")
    s = lax.axis_index("s")

    @pl.when(s == 0)
    def _():
        pltpu.sync_copy(zpool, pool_s)
        pltpu.sync_copy(zcnt, cnt_s)

    pltpu.sync_copy(ones_h, ones_v)
    plsc.subcore_barrier()

    def step(t, _):
        base = s * NODE_T + t * CE
        pltpu.sync_copy(batchp.at[pl.ds(base, CE)], bidx)
        pltpu.sync_copy(h.at[c, pl.ds(base, CE)], rows_v)
        pltpu.sync_copy(rows_v, pool_s.at[bidx], add=True)
        pltpu.sync_copy(ones_v, cnt_s.at[bidx], add=True)
        return _

    lax.fori_loop(0, NODE_T // CE, step, 0)
    plsc.subcore_barrier()

    @pl.when(s == 0)
    def _():
        pltpu.sync_copy(pool_s, outp.at[c])
        pltpu.sync_copy(cnt_s, outc.at[c])


def _pool(h, batchp, ones_h, zpool, zcnt):
    return pl.kernel(
        _pool_body,
        out_type=(jax.ShapeDtypeStruct((NC, PB, 32), jnp.float32),
                  jax.ShapeDtypeStruct((NC, PB, 16), jnp.float32)),
        mesh=_MESH,
        compiler_params=pltpu.CompilerParams(use_tc_tiling_on_sc=False),
        scratch_types=[
            pltpu.VMEM_SHARED((PB, 32), jnp.float32),
            pltpu.VMEM_SHARED((PB, 16), jnp.float32),
            pltpu.VMEM((CE,), jnp.int32),
            pltpu.VMEM((CE, 32), jnp.float32),
            pltpu.VMEM((CE, 16), jnp.float32),
            pltpu.SemaphoreType.DMA,
        ],
    )(h, batchp, ones_h, zpool, zcnt)


def _mlp_body(eps_ref, h_ref, agg_ref, w1_ref, b1_ref, w2_ref, b2_ref, out_ref):
    scale = 1.0 + eps_ref[0]
    z0 = scale * h_ref[0] + agg_ref[0]
    z1 = scale * h_ref[1] + agg_ref[1]
    y = jnp.dot(z0, w1_ref[...][:32, :], preferred_element_type=jnp.float32)
    y += jnp.dot(z1, w1_ref[...][32:, :], preferred_element_type=jnp.float32)
    y = jnp.maximum(y + b1_ref[...], 0.0)
    y = jnp.dot(y, w2_ref[...], preferred_element_type=jnp.float32) + b2_ref[...]
    y = jnp.maximum(y, 0.0)
    out_ref[0] = y[:, :32]
    out_ref[1] = y[:, 32:]


def _mlp_layer(eps_l, h, agg, w1, b1, w2, b2):
    grid = (N_PAD // N_BLK,)
    return pl.pallas_call(
        _mlp_body,
        grid=grid,
        in_specs=[
            pl.BlockSpec(memory_space=pltpu.SMEM),
            pl.BlockSpec((NC, N_BLK, 32), lambda i: (0, i, 0)),
            pl.BlockSpec((NC, N_BLK, 32), lambda i: (0, i, 0)),
            pl.BlockSpec((64, 64), lambda i: (0, 0)),
            pl.BlockSpec((64,), lambda i: (0,)),
            pl.BlockSpec((64, 64), lambda i: (0, 0)),
            pl.BlockSpec((64,), lambda i: (0,)),
        ],
        out_specs=pl.BlockSpec((NC, N_BLK, 32), lambda i: (0, i, 0)),
        out_shape=jax.ShapeDtypeStruct((NC, N_PAD, 32), jnp.float32),
    )(eps_l.reshape(1), h, agg, w1, b1, w2, b2)


def _head_body(p0_ref, p1_ref, cnt_ref, w1_ref, b1_ref, w2f_ref, sel_ref,
               b2_ref, out_ref):
    cnt = jnp.maximum(cnt_ref[...][:, 0:1], 1.0)
    g = jnp.concatenate([p0_ref[...], p1_ref[...]], axis=1) / cnt
    t = jnp.maximum(jnp.dot(g, w1_ref[...], preferred_element_type=jnp.float32)
                    + b1_ref[...], 0.0)
    u = t * w2f_ref[...]
    out_ref[...] = jnp.dot(u, sel_ref[...],
                           preferred_element_type=jnp.float32) + b2_ref[...]


def _head(p0, p1, cnt, w1cat, b1cat, w2flat, sel, b2row):
    P = sel.shape[1]
    return pl.pallas_call(
        _head_body,
        out_shape=jax.ShapeDtypeStruct((B, P), jnp.float32),
    )(p0, p1, cnt, w1cat, b1cat, w2flat, sel, b2row)


def kernel(x, edge_index, batch, embed, CW1, Cb1, CW2, Cb2, eps, HW1, Hb1,
           HW2, Hb2):
    x = x.astype(jnp.int32)
    src = edge_index[0].astype(jnp.int32)
    dst = edge_index[1].astype(jnp.int32)
    batch = batch.astype(jnp.int32)

    x_pad = jnp.pad(x, (0, N_PAD - N))
    x2 = jnp.stack([x_pad, x_pad + V])
    table = jnp.concatenate([embed[:, :32], embed[:, 32:]], axis=0)

    src_pad = jnp.pad(src, (0, E_PAD - E))
    src2 = jnp.stack([src_pad, src_pad + N_PAD])
    dstp = jnp.pad(dst, (0, E_PAD - E), constant_values=N_PAD)
    zrows = jnp.zeros((AGG_ROWS, 32), jnp.float32)

    batchp = jnp.pad(batch, (0, N_PAD - N), constant_values=B)
    ones_h = jnp.ones((CE, 16), jnp.float32)
    zpool = jnp.zeros((PB, 32), jnp.float32)
    zcnt = jnp.zeros((PB, 16), jnp.float32)

    h = _embed(x2, table)
    for l in range(3):
        agg = _agg(h.reshape(NC * N_PAD, 32), src2, dstp, zrows)
        h = _mlp_layer(eps[l], h, agg, CW1[l], Cb1[l], CW2[l], Cb2[l])

    pool, cnt = _pool(h, batchp, ones_h, zpool, zcnt)

    P = HW1.shape[0]
    w1cat = HW1.transpose(1, 0, 2).reshape(H, P * H)
    b1cat = Hb1.reshape(P * H)
    w2flat = HW2[:, :, 0].reshape(P * H)
    sel = jnp.repeat(jnp.eye(P, dtype=jnp.float32), H, axis=0)
    b2row = Hb2[:, 0]
    return _head(pool[0, :B], pool[1, :B], cnt[0, :B], w1cat, b1cat, w2flat,
                 sel, b2row)


# speedup vs baseline: 8.7956x; 2.2633x over previous
import functools

import jax
import jax.numpy as jnp
from jax import lax
from jax.experimental import pallas as pl
from jax.experimental.pallas import tpu as pltpu
from jax.experimental.pallas import tpu_sc as plsc

N = 50000
E = 800000
B = 256
H = 64
V = 51

NC = 2
NS = 16
CE = 128

N_BLK = 1024
N_PAD = 51200
NODE_T = N_PAD // NS
NBUF = 4
EDGE_C = 392
E_T = EDGE_C * CE
E_PAD = NS * E_T
AGG_ROWS = N_PAD + CE
ZERO_T = AGG_ROWS // NS
PB = B + 8

_MESH = plsc.VectorSubcoreMesh(core_axis_name="c", subcore_axis_name="s")


def _embed_body(x2, table, out, idx_v, rows_v, sem):
    c = lax.axis_index("c")
    s = lax.axis_index("s")

    def step(t, _):
        base = s * NODE_T + t * CE
        pltpu.sync_copy(x2.at[c, pl.ds(base, CE)], idx_v)
        pltpu.async_copy(table.at[idx_v], rows_v, sem).wait()
        pltpu.sync_copy(rows_v, out.at[c, pl.ds(base, CE)])
        return _

    lax.fori_loop(0, NODE_T // CE, step, 0)


def _embed(x2, table):
    return pl.kernel(
        _embed_body,
        out_type=jax.ShapeDtypeStruct((NC, N_PAD, 32), jnp.float32),
        mesh=_MESH,
        compiler_params=pltpu.CompilerParams(use_tc_tiling_on_sc=False),
        scratch_types=[
            pltpu.VMEM((CE,), jnp.int32),
            pltpu.VMEM((CE, 32), jnp.float32),
            pltpu.SemaphoreType.DMA,
        ],
    )(x2, table)


def _agg_body(h2d, src3, dst3, zrows, out, agg_s, sidx8, didx8, rows,
              semi, semg, sems):
    c = lax.axis_index("c")
    s = lax.axis_index("s")
    pltpu.sync_copy(zrows.at[pl.ds(s * ZERO_T, ZERO_T)],
                    agg_s.at[pl.ds(s * ZERO_T, ZERO_T)])
    plsc.subcore_barrier()

    sbase = (c * NS + s) * E_T
    dbase = s * E_T

    def fire_idx(t, slot):
        pltpu.async_copy(src3.at[pl.ds(sbase + t * CE, CE)], sidx8.at[slot],
                         semi.at[slot])
        pltpu.async_copy(dst3.at[pl.ds(dbase + t * CE, CE)], didx8.at[slot],
                         semi.at[slot])

    def wait_idx(slot):
        pltpu.make_async_copy(dst3.at[pl.ds(0, CE)], sidx8.at[slot],
                              semi.at[slot]).wait()
        pltpu.make_async_copy(dst3.at[pl.ds(0, CE)], didx8.at[slot],
                              semi.at[slot]).wait()

    def fire_gather(slot):
        pltpu.async_copy(h2d.at[sidx8.at[slot]], rows.at[slot], semg.at[slot])

    def wait_gather(slot):
        pltpu.make_async_copy(h2d.at[pl.ds(0, CE)], rows.at[slot],
                              semg.at[slot]).wait()

    def fire_scatter(slot):
        pltpu.async_copy(rows.at[slot], agg_s.at[didx8.at[slot]],
                         sems.at[slot], add=True)

    def wait_scatter(slot):
        pltpu.make_async_copy(h2d.at[pl.ds(0, CE)], rows.at[slot],
                              sems.at[slot]).wait()

    fire_idx(0, 0)
    fire_idx(1, 1)
    wait_idx(0)
    fire_gather(0)

    def step(jj, carry):
        for b in range(4):
            i = jj * 4 + b

            @pl.when(i >= 2)
            def _stage_a(b=b):
                wait_scatter((b + 2) % 4)

            @pl.when(i + 2 < EDGE_C)
            def _stage_a2(i=i, b=b):
                fire_idx(i + 2, (b + 2) % 4)

            @pl.when(i + 1 < EDGE_C)
            def _stage_b(i=i, b=b):
                wait_idx((b + 1) % 4)
                fire_gather((b + 1) % 4)

            wait_gather(b)
            fire_scatter(b)
        return carry

    lax.fori_loop(0, EDGE_C // 4, step, 0)
    for t in range(EDGE_C - 2, EDGE_C):
        wait_scatter(t % 4)
    plsc.subcore_barrier()
    pltpu.sync_copy(agg_s.at[pl.ds(s * NODE_T, NODE_T)],
                    out.at[c, pl.ds(s * NODE_T, NODE_T)])


def _agg(h2d, src3, dst3, zrows):
    return pl.kernel(
        _agg_body,
        out_type=jax.ShapeDtypeStruct((NC, N_PAD, 32), jnp.float32),
        mesh=_MESH,
        compiler_params=pltpu.CompilerParams(use_tc_tiling_on_sc=False),
        scratch_types=[
            pltpu.VMEM_SHARED((AGG_ROWS, 32), jnp.float32),
            pltpu.VMEM((4, CE), jnp.int32),
            pltpu.VMEM((4, CE), jnp.int32),
            pltpu.VMEM((4, CE, 32), jnp.float32),
            pltpu.SemaphoreType.DMA((4,)),
            pltpu.SemaphoreType.DMA((4,)),
            pltpu.SemaphoreType.DMA((4,)),
        ],
    )(h2d, src3, dst3, zrows)


def _pool_body(h, batchp, ones_h, zpool, zcnt, outp, outc,
               pool_s, cnt_s, bidx, rows_v, ones_v, sem):
    c = lax.axis_index("c")
    s = lax.axis_index("s")

    @pl.when(s == 0)
    def _():
        pltpu.sync_copy(zpool, pool_s)
        pltpu.sync_copy(zcnt, cnt_s)

    pltpu.sync_copy(ones_h, ones_v)
    plsc.subcore_barrier()

    def step(t, _):
        base = s * NODE_T + t * CE
        pltpu.sync_copy(batchp.at[pl.ds(base, CE)], bidx)
        pltpu.sync_copy(h.at[c, pl.ds(base, CE)], rows_v)
        pltpu.sync_copy(rows_v, pool_s.at[bidx], add=True)
        pltpu.sync_copy(ones_v, cnt_s.at[bidx], add=True)
        return _

    lax.fori_loop(0, NODE_T // CE, step, 0)
    plsc.subcore_barrier()

    @pl.when(s == 0)
    def _():
        pltpu.sync_copy(pool_s, outp.at[c])
        pltpu.sync_copy(cnt_s, outc.at[c])


def _pool(h, batchp, ones_h, zpool, zcnt):
    return pl.kernel(
        _pool_body,
        out_type=(jax.ShapeDtypeStruct((NC, PB, 32), jnp.float32),
                  jax.ShapeDtypeStruct((NC, PB, 16), jnp.float32)),
        mesh=_MESH,
        compiler_params=pltpu.CompilerParams(use_tc_tiling_on_sc=False),
        scratch_types=[
            pltpu.VMEM_SHARED((PB, 32), jnp.float32),
            pltpu.VMEM_SHARED((PB, 16), jnp.float32),
            pltpu.VMEM((CE,), jnp.int32),
            pltpu.VMEM((CE, 32), jnp.float32),
            pltpu.VMEM((CE, 16), jnp.float32),
            pltpu.SemaphoreType.DMA,
        ],
    )(h, batchp, ones_h, zpool, zcnt)


def _mlp_body(eps_ref, h_ref, agg_ref, w1_ref, b1_ref, w2_ref, b2_ref, out_ref):
    scale = 1.0 + eps_ref[0]
    z0 = scale * h_ref[0] + agg_ref[0]
    z1 = scale * h_ref[1] + agg_ref[1]
    y = jnp.dot(z0, w1_ref[...][:32, :], preferred_element_type=jnp.float32)
    y += jnp.dot(z1, w1_ref[...][32:, :], preferred_element_type=jnp.float32)
    y = jnp.maximum(y + b1_ref[...], 0.0)
    y = jnp.dot(y, w2_ref[...], preferred_element_type=jnp.float32) + b2_ref[...]
    y = jnp.maximum(y, 0.0)
    out_ref[0] = y[:, :32]
    out_ref[1] = y[:, 32:]


def _mlp_layer(eps_l, h, agg, w1, b1, w2, b2):
    grid = (N_PAD // N_BLK,)
    return pl.pallas_call(
        _mlp_body,
        grid=grid,
        in_specs=[
            pl.BlockSpec(memory_space=pltpu.SMEM),
            pl.BlockSpec((NC, N_BLK, 32), lambda i: (0, i, 0)),
            pl.BlockSpec((NC, N_BLK, 32), lambda i: (0, i, 0)),
            pl.BlockSpec((64, 64), lambda i: (0, 0)),
            pl.BlockSpec((64,), lambda i: (0,)),
            pl.BlockSpec((64, 64), lambda i: (0, 0)),
            pl.BlockSpec((64,), lambda i: (0,)),
        ],
        out_specs=pl.BlockSpec((NC, N_BLK, 32), lambda i: (0, i, 0)),
        out_shape=jax.ShapeDtypeStruct((NC, N_PAD, 32), jnp.float32),
    )(eps_l.reshape(1), h, agg, w1, b1, w2, b2)


def _head_body(p0_ref, p1_ref, cnt_ref, w1_ref, b1_ref, w2f_ref, sel_ref,
               b2_ref, out_ref):
    cnt = jnp.maximum(cnt_ref[...][:, 0:1], 1.0)
    g = jnp.concatenate([p0_ref[...], p1_ref[...]], axis=1) / cnt
    t = jnp.maximum(jnp.dot(g, w1_ref[...], preferred_element_type=jnp.float32)
                    + b1_ref[...], 0.0)
    u = t * w2f_ref[...]
    out_ref[...] = jnp.dot(u, sel_ref[...],
                           preferred_element_type=jnp.float32) + b2_ref[...]


def _head(p0, p1, cnt, w1cat, b1cat, w2flat, sel, b2row):
    P = sel.shape[1]
    return pl.pallas_call(
        _head_body,
        out_shape=jax.ShapeDtypeStruct((B, P), jnp.float32),
    )(p0, p1, cnt, w1cat, b1cat, w2flat, sel, b2row)


def kernel(x, edge_index, batch, embed, CW1, Cb1, CW2, Cb2, eps, HW1, Hb1,
           HW2, Hb2):
    x = x.astype(jnp.int32)
    src = edge_index[0].astype(jnp.int32)
    dst = edge_index[1].astype(jnp.int32)
    batch = batch.astype(jnp.int32)

    x_pad = jnp.pad(x, (0, N_PAD - N))
    x2 = jnp.stack([x_pad, x_pad + V])
    table = jnp.concatenate([embed[:, :32], embed[:, 32:]], axis=0)

    src_pad = jnp.pad(src, (0, E_PAD - E))
    src3 = jnp.stack([src_pad, src_pad + N_PAD]).reshape(NC * E_PAD)
    dst3 = jnp.pad(dst, (0, E_PAD - E), constant_values=N_PAD)
    zrows = jnp.zeros((AGG_ROWS, 32), jnp.float32)

    batchp = jnp.pad(batch, (0, N_PAD - N), constant_values=B)
    ones_h = jnp.ones((CE, 16), jnp.float32)
    zpool = jnp.zeros((PB, 32), jnp.float32)
    zcnt = jnp.zeros((PB, 16), jnp.float32)

    h = _embed(x2, table)
    for l in range(3):
        agg = _agg(h.reshape(NC * N_PAD, 32), src3, dst3, zrows)
        h = _mlp_layer(eps[l], h, agg, CW1[l], Cb1[l], CW2[l], Cb2[l])

    pool, cnt = _pool(h, batchp, ones_h, zpool, zcnt)

    P = HW1.shape[0]
    w1cat = HW1.transpose(1, 0, 2).reshape(H, P * H)
    b1cat = Hb1.reshape(P * H)
    w2flat = HW2[:, :, 0].reshape(P * H)
    sel = jnp.repeat(jnp.eye(P, dtype=jnp.float32), H, axis=0)
    b2row = Hb2[:, 0]
    return _head(pool[0, :B], pool[1, :B], cnt[0, :B], w1cat, b1cat, w2flat,
                 sel, b2row)
